# baseline (speedup 1.0000x reference)
import functools

import jax
import jax.numpy as jnp
from jax import lax
from jax.experimental import pallas as pl
from jax.experimental.pallas import tpu as pltpu
from jax.experimental.pallas import tpu_sc as plsc

B, C, S, L = 64, 200, 6, 9
D_TOK, D_NODE, H, D_DEC = 128, 128, 128, 512
NODE_VOCAB = 512
N = B * C
NCTX = 2 * N
CHUNK_CTX = 16
ROWS_PER_CHUNK = CHUNK_CTX * S
N_CHUNKS = NCTX // CHUNK_CTX
NW = 32
CHUNKS_PER_W = N_CHUNKS // NW
LANES = 16
TILE = 256
G4 = 4 * H


def _sc_pool_body(idx_hbm, mask_hbm, table_hbm, out_hbm,
                  idx_all, mask_v, rows_v, acc_v, gsem, msem, osem):
    wid = lax.axis_index("s") * 2 + lax.axis_index("c")
    base = wid * CHUNKS_PER_W

    pltpu.sync_copy(idx_hbm.at[wid], idx_all)

    def issue(k, b):
        pltpu.async_copy(mask_hbm.at[base + k], mask_v.at[b], msem.at[b])
        pltpu.async_copy(table_hbm.at[idx_all.at[k]], rows_v.at[b], gsem.at[b])

    issue(0, 0)
    issue(1, 1)

    def outer(jj, _):
        for b in range(2):
            k = 2 * jj + b
            pltpu.make_async_copy(mask_hbm.at[0], mask_v.at[b], msem.at[b]).wait()
            pltpu.make_async_copy(table_hbm.at[idx_all.at[0]], rows_v.at[b], gsem.at[b]).wait()

            @pl.when(k >= 2)
            def _():
                pltpu.make_async_copy(
                    acc_v.at[b], out_hbm.at[pl.ds(0, CHUNK_CTX)], osem.at[b]).wait()

            for ci in range(CHUNK_CTX):
                ms = [mask_v[b, ci * S + s, :] for s in range(S)]
                for v in range(D_TOK // LANES):
                    acc = rows_v[b, ci * S + 0, pl.ds(v * LANES, LANES)] * ms[0]
                    for s in range(1, S):
                        acc = acc + rows_v[b, ci * S + s, pl.ds(v * LANES, LANES)] * ms[s]
                    acc_v[b, ci, pl.ds(v * LANES, LANES)] = acc
            pltpu.async_copy(
                acc_v.at[b], out_hbm.at[pl.ds((base + k) * CHUNK_CTX, CHUNK_CTX)],
                osem.at[b])

            @pl.when(k + 2 < CHUNKS_PER_W)
            def _():
                issue(k + 2, b)
        return 0

    lax.fori_loop(0, CHUNKS_PER_W // 2, outer, 0)
    for b in range(2):
        pltpu.make_async_copy(
            acc_v.at[b], out_hbm.at[pl.ds(0, CHUNK_CTX)], osem.at[b]).wait()


def _sc_pool_call(idx_cat, mask_cat, table):
    mesh = plsc.VectorSubcoreMesh(core_axis_name="c", subcore_axis_name="s")
    fn = functools.partial(
        pl.kernel,
        mesh=mesh,
        out_type=jax.ShapeDtypeStruct((NCTX, D_TOK), jnp.float32),
        scratch_types=[
            pltpu.VMEM((CHUNKS_PER_W, ROWS_PER_CHUNK), jnp.int32),
            pltpu.VMEM((2, ROWS_PER_CHUNK, LANES), jnp.float32),
            pltpu.VMEM((2, ROWS_PER_CHUNK, D_TOK), jnp.float32),
            pltpu.VMEM((2, CHUNK_CTX, D_TOK), jnp.float32),
            pltpu.SemaphoreType.DMA((2,)),
            pltpu.SemaphoreType.DMA((2,)),
            pltpu.SemaphoreType.DMA((2,)),
        ],
    )(_sc_pool_body)
    return fn(idx_cat, mask_cat, table)


def _prep_body(emb_ref, wxf_ref, bf_ref, wxb_ref, bb_ref, t_ref):
    emb = emb_ref[...]
    tf = jnp.dot(emb, wxf_ref[...], preferred_element_type=jnp.float32) + bf_ref[...]
    tb = jnp.dot(emb, wxb_ref[...], preferred_element_type=jnp.float32) + bb_ref[...]
    t_ref[...] = jnp.concatenate([tf, tb], axis=1).astype(jnp.bfloat16)


def _prep_call(node_embedding, Wx_f, b_f, Wx_b, b_b):
    return pl.pallas_call(
        _prep_body,
        out_shape=jax.ShapeDtypeStruct((NODE_VOCAB, 2 * G4), jnp.bfloat16),
    )(node_embedding, Wx_f, b_f.reshape(1, G4), Wx_b, b_b.reshape(1, G4))


def _sigmoid(x):
    return 0.5 + 0.5 * jnp.tanh(0.5 * x)


def _lstm_body(idx_ref, len_ref, t_ref, wh_ref, out_ref, oh_ref, g_ref):
    f32 = jnp.float32
    idx = idx_ref[...]
    iota = lax.broadcasted_iota(jnp.int32, (TILE, NODE_VOCAB), 1)
    for p in range(L):
        oh_ref[pl.ds(p * TILE, TILE), :] = (idx[:, p:p + 1] == iota).astype(jnp.bfloat16)
    g_ref[...] = jnp.dot(oh_ref[...], t_ref[...], preferred_element_type=f32)

    lenc = len_ref[...]
    wh = wh_ref[...]
    hf = jnp.zeros((TILE, H), f32)
    cf = jnp.zeros((TILE, H), f32)
    hb = jnp.zeros((TILE, H), f32)
    cb = jnp.zeros((TILE, H), f32)

    def cell(gates, h, c, mask):
        i = _sigmoid(gates[:, 0:H])
        f = _sigmoid(gates[:, H:2 * H])
        g = jnp.tanh(gates[:, 2 * H:3 * H])
        o = _sigmoid(gates[:, 3 * H:4 * H])
        c_new = f * c + i * g
        h_new = o * jnp.tanh(c_new)
        return jnp.where(mask, h_new, h), jnp.where(mask, c_new, c)

    for t in range(L):
        hcat = jnp.concatenate([hf, hb], axis=1).astype(jnp.bfloat16)
        rec = jnp.dot(hcat, wh, preferred_element_type=f32)
        gf = g_ref[pl.ds(t * TILE, TILE), 0:G4] + rec[:, 0:G4]
        gb = g_ref[pl.ds((L - 1 - t) * TILE, TILE), G4:2 * G4] + rec[:, G4:2 * G4]
        hf, cf = cell(gf, hf, cf, t < lenc)
        hb, cb = cell(gb, hb, cb, (L - 1 - t) < lenc)

    out_ref[...] = jnp.concatenate([hf, hb], axis=1).astype(jnp.bfloat16)


def _lstm_call(idx, lens, tcat, whcat):
    row = lambda i: (i, 0)
    rep = lambda i: (0, 0)
    return pl.pallas_call(
        _lstm_body,
        grid=(N // TILE,),
        in_specs=[
            pl.BlockSpec((TILE, L), row),
            pl.BlockSpec((TILE, 1), row),
            pl.BlockSpec((NODE_VOCAB, 2 * G4), rep),
            pl.BlockSpec((2 * H, 2 * G4), rep),
        ],
        out_specs=pl.BlockSpec((TILE, 2 * H), row),
        out_shape=jax.ShapeDtypeStruct((N, 2 * H), jnp.bfloat16),
        scratch_shapes=[
            pltpu.VMEM((L * TILE, NODE_VOCAB), jnp.bfloat16),
            pltpu.VMEM((L * TILE, 2 * G4), jnp.float32),
        ],
    )(idx, lens, tcat, whcat)


def _gemm_body(sa_ref, ta_ref, h_ref, cvm_ref, w_ref, out_ref):
    f32 = jnp.float32
    bf16 = jnp.bfloat16
    w = w_ref[...]
    cvm = cvm_ref[...].astype(bf16)
    out = (jnp.dot(sa_ref[...].astype(bf16), w[0:D_TOK], preferred_element_type=f32)
           + jnp.dot(h_ref[...] * cvm, w[D_TOK:D_TOK + 2 * H], preferred_element_type=f32)
           + jnp.dot(ta_ref[...].astype(bf16), w[D_TOK + 2 * H:], preferred_element_type=f32))
    out_ref[...] = jnp.tanh(out)


def _gemm_call(src_agg, tgt_agg, hcat, cvm, wctx):
    row = lambda i: (i, 0)
    rep = lambda i: (0, 0)
    return pl.pallas_call(
        _gemm_body,
        grid=(N // TILE,),
        in_specs=[
            pl.BlockSpec((TILE, D_TOK), row),
            pl.BlockSpec((TILE, D_TOK), row),
            pl.BlockSpec((TILE, 2 * H), row),
            pl.BlockSpec((TILE, 1), row),
            pl.BlockSpec((2 * (D_TOK + H), D_DEC), rep),
        ],
        out_specs=pl.BlockSpec((TILE, D_DEC), row),
        out_shape=jax.ShapeDtypeStruct((N, D_DEC), jnp.float32),
    )(src_agg, tgt_agg, hcat, cvm, wctx)


def kernel(source_subtoken_indices, node_indices, target_subtoken_indices,
           source_subtoken_lengths, node_lengths, target_subtoken_lengths,
           context_valid_mask, subtoken_embedding, node_embedding,
           Wx_f, Wh_f, b_f, Wx_b, Wh_b, b_b, W_ctx):
    src_idx = source_subtoken_indices.reshape(N, S)
    tgt_idx = target_subtoken_indices.reshape(N, S)
    idx_cat = jnp.concatenate([src_idx, tgt_idx], axis=0).reshape(
        NW, CHUNKS_PER_W, ROWS_PER_CHUNK)
    ar = jnp.arange(S)[None, :]
    src_mask = (ar < source_subtoken_lengths.reshape(N, 1)).astype(jnp.float32)
    tgt_mask = (ar < target_subtoken_lengths.reshape(N, 1)).astype(jnp.float32)
    mask_cat = jnp.concatenate([src_mask, tgt_mask], axis=0).reshape(NCTX * S, 1)
    mask_cat = jnp.broadcast_to(mask_cat, (NCTX * S, LANES)).reshape(
        N_CHUNKS, ROWS_PER_CHUNK, LANES)

    nidx = node_indices.reshape(N, L)
    lens = node_lengths.reshape(N, 1)
    whcat = jnp.zeros((2 * H, 2 * G4), jnp.float32)
    whcat = whcat.at[0:H, 0:G4].set(Wh_f).at[H:2 * H, G4:2 * G4].set(Wh_b)
    whcat = whcat.astype(jnp.bfloat16)
    wctx = W_ctx.astype(jnp.bfloat16)

    pooled = _sc_pool_call(idx_cat, mask_cat, subtoken_embedding)

    tcat = _prep_call(node_embedding, Wx_f, b_f, Wx_b, b_b)
    hcat = _lstm_call(nidx, lens, tcat, whcat)
    out = _gemm_call(pooled[:N], pooled[N:], hcat,
                     context_valid_mask.reshape(N, 1), wctx)
    return out.reshape(B, C, D_DEC)


# speedup vs baseline: 1.2252x; 1.2252x over previous
import functools

import jax
import jax.numpy as jnp
from jax import lax
from jax.experimental import pallas as pl
from jax.experimental.pallas import tpu as pltpu
from jax.experimental.pallas import tpu_sc as plsc

B, C, S, L = 64, 200, 6, 9
D_TOK, D_NODE, H, D_DEC = 128, 128, 128, 512
NODE_VOCAB = 512
N = B * C
NCTX = 2 * N
CHUNK_CTX = 16
ROWS_PER_CHUNK = CHUNK_CTX * S
N_CHUNKS = NCTX // CHUNK_CTX
NW = 32
CHUNKS_PER_W = N_CHUNKS // NW
LANES = 16
MROW = ROWS_PER_CHUNK * LANES
TILE = 512
G4 = 4 * H


def _sc_pool_body(idx_hbm, mask_hbm, table_hbm, out_hbm,
                  idx_all, mask_v, rows_v, acc_v, gsem, msem, osem):
    wid = lax.axis_index("s") * 2 + lax.axis_index("c")
    base = wid * CHUNKS_PER_W

    pltpu.sync_copy(idx_hbm.at[wid], idx_all)

    def issue(k, b):
        pltpu.async_copy(mask_hbm.at[base + k], mask_v.at[b], msem.at[b])
        pltpu.async_copy(
            table_hbm.at[idx_all.at[pl.ds(k * ROWS_PER_CHUNK, ROWS_PER_CHUNK)]],
            rows_v.at[b], gsem.at[b])

    issue(0, 0)
    issue(1, 1)

    def outer(jj, _):
        for b in range(2):
            k = 2 * jj + b
            pltpu.make_async_copy(mask_hbm.at[0], mask_v.at[b], msem.at[b]).wait()
            pltpu.make_async_copy(
                table_hbm.at[idx_all.at[pl.ds(0, ROWS_PER_CHUNK)]],
                rows_v.at[b], gsem.at[b]).wait()

            @pl.when(k >= 2)
            def _():
                pltpu.make_async_copy(
                    acc_v.at[b], out_hbm.at[pl.ds(0, CHUNK_CTX)], osem.at[b]).wait()

            for ci in range(CHUNK_CTX):
                ms = [mask_v[b, pl.ds((ci * S + s) * LANES, LANES)] for s in range(S)]
                for v in range(D_TOK // LANES):
                    acc = rows_v[b, ci * S + 0, pl.ds(v * LANES, LANES)] * ms[0]
                    for s in range(1, S):
                        acc = acc + rows_v[b, ci * S + s, pl.ds(v * LANES, LANES)] * ms[s]
                    acc_v[b, ci, pl.ds(v * LANES, LANES)] = acc
            pltpu.async_copy(
                acc_v.at[b], out_hbm.at[pl.ds((base + k) * CHUNK_CTX, CHUNK_CTX)],
                osem.at[b])

            @pl.when(k + 2 < CHUNKS_PER_W)
            def _():
                issue(k + 2, b)
        return 0

    lax.fori_loop(0, CHUNKS_PER_W // 2, outer, 0)
    for b in range(2):
        pltpu.make_async_copy(
            acc_v.at[b], out_hbm.at[pl.ds(0, CHUNK_CTX)], osem.at[b]).wait()


def _sc_pool_call(idx_cat, mask_cat, table):
    mesh = plsc.VectorSubcoreMesh(core_axis_name="c", subcore_axis_name="s")
    fn = functools.partial(
        pl.kernel,
        mesh=mesh,
        out_type=jax.ShapeDtypeStruct((NCTX, D_TOK), jnp.float32),
        scratch_types=[
            pltpu.VMEM((CHUNKS_PER_W * ROWS_PER_CHUNK,), jnp.int32),
            pltpu.VMEM((2, MROW), jnp.float32),
            pltpu.VMEM((2, ROWS_PER_CHUNK, D_TOK), jnp.float32),
            pltpu.VMEM((2, CHUNK_CTX, D_TOK), jnp.float32),
            pltpu.SemaphoreType.DMA((2,)),
            pltpu.SemaphoreType.DMA((2,)),
            pltpu.SemaphoreType.DMA((2,)),
        ],
    )(_sc_pool_body)
    return fn(idx_cat, mask_cat, table)


def _prep_body(emb_ref, wxf_ref, bf_ref, wxb_ref, bb_ref, t_ref):
    emb = emb_ref[...]
    tf = jnp.dot(emb, wxf_ref[...], preferred_element_type=jnp.float32) + bf_ref[...]
    tb = jnp.dot(emb, wxb_ref[...], preferred_element_type=jnp.float32) + bb_ref[...]
    t_ref[...] = jnp.concatenate([tf, tb], axis=1).astype(jnp.bfloat16)


def _prep_call(node_embedding, Wx_f, b_f, Wx_b, b_b):
    return pl.pallas_call(
        _prep_body,
        out_shape=jax.ShapeDtypeStruct((NODE_VOCAB, 2 * G4), jnp.bfloat16),
    )(node_embedding, Wx_f, b_f.reshape(1, G4), Wx_b, b_b.reshape(1, G4))


def _sigmoid(x):
    return 0.5 + 0.5 * jnp.tanh(0.5 * x)


def _lstm_body(idx_ref, len_ref, t_ref, wh_ref, out_ref):
    f32 = jnp.float32
    idx = idx_ref[...]
    iota = lax.broadcasted_iota(jnp.int32, (TILE, NODE_VOCAB), 1)
    lenc = len_ref[...]
    tcat = t_ref[...]
    wh = wh_ref[...]
    hf = jnp.zeros((TILE, H), f32)
    cf = jnp.zeros((TILE, H), f32)
    hb = jnp.zeros((TILE, H), f32)
    cb = jnp.zeros((TILE, H), f32)

    def cell(gates, h, c, mask):
        i = _sigmoid(gates[:, 0:H])
        f = _sigmoid(gates[:, H:2 * H])
        g = jnp.tanh(gates[:, 2 * H:3 * H])
        o = _sigmoid(gates[:, 3 * H:4 * H])
        c_new = f * c + i * g
        h_new = o * jnp.tanh(c_new)
        return jnp.where(mask, h_new, h), jnp.where(mask, c_new, c)

    for t in range(L):
        oh_f = (idx[:, t:t + 1] == iota).astype(jnp.bfloat16)
        oh_b = (idx[:, L - 1 - t:L - t] == iota).astype(jnp.bfloat16)
        hcat = jnp.concatenate([hf, hb], axis=1).astype(jnp.bfloat16)
        rec = jnp.dot(hcat, wh, preferred_element_type=f32)
        gf = jnp.dot(oh_f, tcat[:, 0:G4], preferred_element_type=f32) + rec[:, 0:G4]
        gb = jnp.dot(oh_b, tcat[:, G4:2 * G4], preferred_element_type=f32) + rec[:, G4:2 * G4]
        hf, cf = cell(gf, hf, cf, t < lenc)
        hb, cb = cell(gb, hb, cb, (L - 1 - t) < lenc)

    out_ref[...] = jnp.concatenate([hf, hb], axis=1).astype(jnp.bfloat16)


def _lstm_call(idx, lens, tcat, whcat):
    row = lambda i: (i, 0)
    rep = lambda i: (0, 0)
    return pl.pallas_call(
        _lstm_body,
        grid=(N // TILE,),
        in_specs=[
            pl.BlockSpec((TILE, L), row),
            pl.BlockSpec((TILE, 1), row),
            pl.BlockSpec((NODE_VOCAB, 2 * G4), rep),
            pl.BlockSpec((2 * H, 2 * G4), rep),
        ],
        out_specs=pl.BlockSpec((TILE, 2 * H), row),
        out_shape=jax.ShapeDtypeStruct((N, 2 * H), jnp.bfloat16),
    )(idx, lens, tcat, whcat)


def _gemm_body(sa_ref, ta_ref, h_ref, cvm_ref, w_ref, out_ref):
    f32 = jnp.float32
    bf16 = jnp.bfloat16
    w = w_ref[...]
    cvm = cvm_ref[...].astype(bf16)
    out = (jnp.dot(sa_ref[...].astype(bf16), w[0:D_TOK], preferred_element_type=f32)
           + jnp.dot(h_ref[...] * cvm, w[D_TOK:D_TOK + 2 * H], preferred_element_type=f32)
           + jnp.dot(ta_ref[...].astype(bf16), w[D_TOK + 2 * H:], preferred_element_type=f32))
    out_ref[...] = jnp.tanh(out)


def _gemm_call(src_agg, tgt_agg, hcat, cvm, wctx):
    row = lambda i: (i, 0)
    rep = lambda i: (0, 0)
    return pl.pallas_call(
        _gemm_body,
        grid=(N // TILE,),
        in_specs=[
            pl.BlockSpec((TILE, D_TOK), row),
            pl.BlockSpec((TILE, D_TOK), row),
            pl.BlockSpec((TILE, 2 * H), row),
            pl.BlockSpec((TILE, 1), row),
            pl.BlockSpec((2 * (D_TOK + H), D_DEC), rep),
        ],
        out_specs=pl.BlockSpec((TILE, D_DEC), row),
        out_shape=jax.ShapeDtypeStruct((N, D_DEC), jnp.float32),
    )(src_agg, tgt_agg, hcat, cvm, wctx)


def kernel(source_subtoken_indices, node_indices, target_subtoken_indices,
           source_subtoken_lengths, node_lengths, target_subtoken_lengths,
           context_valid_mask, subtoken_embedding, node_embedding,
           Wx_f, Wh_f, b_f, Wx_b, Wh_b, b_b, W_ctx):
    src_idx = source_subtoken_indices.reshape(N, S)
    tgt_idx = target_subtoken_indices.reshape(N, S)
    idx_cat = jnp.concatenate([src_idx, tgt_idx], axis=0).reshape(
        NW, CHUNKS_PER_W * ROWS_PER_CHUNK)
    ar = jnp.arange(S)[None, :]
    src_mask = (ar < source_subtoken_lengths.reshape(N, 1)).astype(jnp.float32)
    tgt_mask = (ar < target_subtoken_lengths.reshape(N, 1)).astype(jnp.float32)
    mask_cat = jnp.concatenate([src_mask, tgt_mask], axis=0).reshape(NCTX * S, 1)
    mask_cat = jnp.broadcast_to(mask_cat, (NCTX * S, LANES)).reshape(N_CHUNKS, MROW)

    nidx = node_indices.reshape(N, L)
    lens = node_lengths.reshape(N, 1)
    whcat = jnp.zeros((2 * H, 2 * G4), jnp.float32)
    whcat = whcat.at[0:H, 0:G4].set(Wh_f).at[H:2 * H, G4:2 * G4].set(Wh_b)
    whcat = whcat.astype(jnp.bfloat16)
    wctx = W_ctx.astype(jnp.bfloat16)

    pooled = _sc_pool_call(idx_cat, mask_cat, subtoken_embedding)

    tcat = _prep_call(node_embedding, Wx_f, b_f, Wx_b, b_b)
    hcat = _lstm_call(nidx, lens, tcat, whcat)
    out = _gemm_call(pooled[:N], pooled[N:], hcat,
                     context_valid_mask.reshape(N, 1), wctx)
    return out.reshape(B, C, D_DEC)


# speedup vs baseline: 1.2895x; 1.0525x over previous
import functools

import jax
import jax.numpy as jnp
from jax import lax
from jax.experimental import pallas as pl
from jax.experimental.pallas import tpu as pltpu
from jax.experimental.pallas import tpu_sc as plsc

B, C, S, L = 64, 200, 6, 9
D_TOK, D_NODE, H, D_DEC = 128, 128, 128, 512
NODE_VOCAB = 512
N = B * C
NCTX = 2 * N
CHUNK_CTX = 16
ROWS_PER_CHUNK = CHUNK_CTX * S
N_CHUNKS = NCTX // CHUNK_CTX
NW = 32
CHUNKS_PER_W = N_CHUNKS // NW
LANES = 16
MROW = ROWS_PER_CHUNK * LANES
TILE = 512
G4 = 4 * H


def _sc_pool_body(idx_hbm, mask_hbm, table_hbm, out_hbm,
                  idx_all, mask_v, rows_v, acc_v, gsem, msem, osem):
    wid = lax.axis_index("s") * 2 + lax.axis_index("c")
    base = wid * CHUNKS_PER_W

    pltpu.sync_copy(idx_hbm.at[wid], idx_all)

    def issue(k, b):
        pltpu.async_copy(mask_hbm.at[base + k], mask_v.at[b], msem.at[b])
        pltpu.async_copy(
            table_hbm.at[idx_all.at[pl.ds(k * ROWS_PER_CHUNK, ROWS_PER_CHUNK)]],
            rows_v.at[b], gsem.at[b])

    issue(0, 0)
    issue(1, 1)

    def outer(jj, _):
        for b in range(2):
            k = 2 * jj + b
            pltpu.make_async_copy(mask_hbm.at[0], mask_v.at[b], msem.at[b]).wait()
            pltpu.make_async_copy(
                table_hbm.at[idx_all.at[pl.ds(0, ROWS_PER_CHUNK)]],
                rows_v.at[b], gsem.at[b]).wait()

            @pl.when(k >= 2)
            def _():
                pltpu.make_async_copy(
                    acc_v.at[b], out_hbm.at[pl.ds(0, CHUNK_CTX)], osem.at[b]).wait()

            for ci in range(CHUNK_CTX):
                ms = [mask_v[b, pl.ds((ci * S + s) * LANES, LANES)] for s in range(S)]
                for v in range(D_TOK // LANES):
                    acc = rows_v[b, ci * S + 0, pl.ds(v * LANES, LANES)] * ms[0]
                    for s in range(1, S):
                        acc = acc + rows_v[b, ci * S + s, pl.ds(v * LANES, LANES)] * ms[s]
                    acc_v[b, ci, pl.ds(v * LANES, LANES)] = acc
            pltpu.async_copy(
                acc_v.at[b], out_hbm.at[pl.ds((base + k) * CHUNK_CTX, CHUNK_CTX)],
                osem.at[b])

            @pl.when(k + 2 < CHUNKS_PER_W)
            def _():
                issue(k + 2, b)
        return 0

    lax.fori_loop(0, CHUNKS_PER_W // 2, outer, 0)
    for b in range(2):
        pltpu.make_async_copy(
            acc_v.at[b], out_hbm.at[pl.ds(0, CHUNK_CTX)], osem.at[b]).wait()


def _sc_pool_call(idx_cat, mask_cat, table):
    mesh = plsc.VectorSubcoreMesh(core_axis_name="c", subcore_axis_name="s")
    fn = functools.partial(
        pl.kernel,
        mesh=mesh,
        out_type=jax.ShapeDtypeStruct((NCTX, D_TOK), jnp.float32),
        scratch_types=[
            pltpu.VMEM((CHUNKS_PER_W * ROWS_PER_CHUNK,), jnp.int32),
            pltpu.VMEM((2, MROW), jnp.float32),
            pltpu.VMEM((2, ROWS_PER_CHUNK, D_TOK), jnp.float32),
            pltpu.VMEM((2, CHUNK_CTX, D_TOK), jnp.float32),
            pltpu.SemaphoreType.DMA((2,)),
            pltpu.SemaphoreType.DMA((2,)),
            pltpu.SemaphoreType.DMA((2,)),
        ],
    )(_sc_pool_body)
    return fn(idx_cat, mask_cat, table)


def _prep_body(emb_ref, wxf_ref, bf_ref, wxb_ref, bb_ref, whf_ref, whb_ref,
               wf_ref, wb_ref):
    f32 = jnp.float32
    emb = emb_ref[...]
    lane = lax.broadcasted_iota(jnp.int32, (1, G4), 1)
    sc = jnp.where((lane >= 2 * H) & (lane < 3 * H), 1.0, 0.5).astype(f32)
    tf = (jnp.dot(emb, wxf_ref[...], preferred_element_type=f32) + bf_ref[...]) * sc
    tb = (jnp.dot(emb, wxb_ref[...], preferred_element_type=f32) + bb_ref[...]) * sc
    wf_ref[0:NODE_VOCAB, :] = tf.astype(jnp.bfloat16)
    wf_ref[NODE_VOCAB:NODE_VOCAB + H, :] = (whf_ref[...] * sc).astype(jnp.bfloat16)
    wb_ref[0:NODE_VOCAB, :] = tb.astype(jnp.bfloat16)
    wb_ref[NODE_VOCAB:NODE_VOCAB + H, :] = (whb_ref[...] * sc).astype(jnp.bfloat16)


def _prep_call(node_embedding, Wx_f, b_f, Wx_b, b_b, Wh_f, Wh_b):
    return pl.pallas_call(
        _prep_body,
        out_shape=(
            jax.ShapeDtypeStruct((NODE_VOCAB + H, G4), jnp.bfloat16),
            jax.ShapeDtypeStruct((NODE_VOCAB + H, G4), jnp.bfloat16),
        ),
    )(node_embedding, Wx_f, b_f.reshape(1, G4), Wx_b, b_b.reshape(1, G4),
      Wh_f, Wh_b)


def _lstm_body(idx_ref, len_ref, wf_ref, wb_ref, out_ref):
    f32 = jnp.float32
    bf16 = jnp.bfloat16
    idx = idx_ref[...]
    iota = lax.broadcasted_iota(jnp.int32, (TILE, NODE_VOCAB), 1)
    lenc = len_ref[...]
    wf = wf_ref[...]
    wb = wb_ref[...]
    hf = jnp.zeros((TILE, H), f32)
    cf = jnp.zeros((TILE, H), f32)
    hb = jnp.zeros((TILE, H), f32)
    cb = jnp.zeros((TILE, H), f32)

    def cell(gate, h, c, mask):
        ti = jnp.tanh(gate[:, 0:H])
        tf_ = jnp.tanh(gate[:, H:2 * H])
        g = jnp.tanh(gate[:, 2 * H:3 * H])
        to = jnp.tanh(gate[:, 3 * H:4 * H])
        c_new = 0.5 * ((c + g) + (tf_ * c + ti * g))
        tc = jnp.tanh(c_new)
        h_new = 0.5 * (tc + to * tc)
        return jnp.where(mask, h_new, h), jnp.where(mask, c_new, c)

    for t in range(L):
        oh_f = (idx[:, t:t + 1] == iota).astype(bf16)
        oh_b = (idx[:, L - 1 - t:L - t] == iota).astype(bf16)
        mf = jnp.concatenate([oh_f, hf.astype(bf16)], axis=1)
        mb = jnp.concatenate([oh_b, hb.astype(bf16)], axis=1)
        gf = jnp.dot(mf, wf, preferred_element_type=f32)
        gb = jnp.dot(mb, wb, preferred_element_type=f32)
        hf, cf = cell(gf, hf, cf, t < lenc)
        hb, cb = cell(gb, hb, cb, (L - 1 - t) < lenc)

    out_ref[...] = jnp.concatenate([hf, hb], axis=1).astype(bf16)


def _lstm_call(idx, lens, wf, wb):
    row = lambda i: (i, 0)
    rep = lambda i: (0, 0)
    return pl.pallas_call(
        _lstm_body,
        grid=(N // TILE,),
        in_specs=[
            pl.BlockSpec((TILE, L), row),
            pl.BlockSpec((TILE, 1), row),
            pl.BlockSpec((NODE_VOCAB + H, G4), rep),
            pl.BlockSpec((NODE_VOCAB + H, G4), rep),
        ],
        out_specs=pl.BlockSpec((TILE, 2 * H), row),
        out_shape=jax.ShapeDtypeStruct((N, 2 * H), jnp.bfloat16),
    )(idx, lens, wf, wb)


def _gemm_body(sa_ref, ta_ref, h_ref, cvm_ref, w_ref, out_ref):
    f32 = jnp.float32
    bf16 = jnp.bfloat16
    w = w_ref[...]
    cvm = cvm_ref[...].astype(bf16)
    out = (jnp.dot(sa_ref[...].astype(bf16), w[0:D_TOK], preferred_element_type=f32)
           + jnp.dot(h_ref[...] * cvm, w[D_TOK:D_TOK + 2 * H], preferred_element_type=f32)
           + jnp.dot(ta_ref[...].astype(bf16), w[D_TOK + 2 * H:], preferred_element_type=f32))
    out_ref[...] = jnp.tanh(out)


def _gemm_call(src_agg, tgt_agg, hcat, cvm, wctx):
    row = lambda i: (i, 0)
    rep = lambda i: (0, 0)
    return pl.pallas_call(
        _gemm_body,
        grid=(N // TILE,),
        in_specs=[
            pl.BlockSpec((TILE, D_TOK), row),
            pl.BlockSpec((TILE, D_TOK), row),
            pl.BlockSpec((TILE, 2 * H), row),
            pl.BlockSpec((TILE, 1), row),
            pl.BlockSpec((2 * (D_TOK + H), D_DEC), rep),
        ],
        out_specs=pl.BlockSpec((TILE, D_DEC), row),
        out_shape=jax.ShapeDtypeStruct((N, D_DEC), jnp.float32),
    )(src_agg, tgt_agg, hcat, cvm, wctx)


def kernel(source_subtoken_indices, node_indices, target_subtoken_indices,
           source_subtoken_lengths, node_lengths, target_subtoken_lengths,
           context_valid_mask, subtoken_embedding, node_embedding,
           Wx_f, Wh_f, b_f, Wx_b, Wh_b, b_b, W_ctx):
    src_idx = source_subtoken_indices.reshape(N, S)
    tgt_idx = target_subtoken_indices.reshape(N, S)
    idx_cat = jnp.concatenate([src_idx, tgt_idx], axis=0).reshape(
        NW, CHUNKS_PER_W * ROWS_PER_CHUNK)
    ar = jnp.arange(S)[None, :]
    src_mask = (ar < source_subtoken_lengths.reshape(N, 1)).astype(jnp.float32)
    tgt_mask = (ar < target_subtoken_lengths.reshape(N, 1)).astype(jnp.float32)
    mask_cat = jnp.concatenate([src_mask, tgt_mask], axis=0).reshape(NCTX * S, 1)
    mask_cat = jnp.broadcast_to(mask_cat, (NCTX * S, LANES)).reshape(N_CHUNKS, MROW)

    nidx = node_indices.reshape(N, L)
    lens = node_lengths.reshape(N, 1)
    wctx = W_ctx.astype(jnp.bfloat16)

    pooled = _sc_pool_call(idx_cat, mask_cat, subtoken_embedding)

    wf, wb = _prep_call(node_embedding, Wx_f, b_f, Wx_b, b_b, Wh_f, Wh_b)
    hcat = _lstm_call(nidx, lens, wf, wb)
    out = _gemm_call(pooled[:N], pooled[N:], hcat,
                     context_valid_mask.reshape(N, 1), wctx)
    return out.reshape(B, C, D_DEC)


# speedup vs baseline: 1.6628x; 1.2896x over previous
import functools

import jax
import jax.numpy as jnp
from jax import lax
from jax.experimental import pallas as pl
from jax.experimental.pallas import tpu as pltpu
from jax.experimental.pallas import tpu_sc as plsc

B, C, S, L = 64, 200, 6, 9
D_TOK, D_NODE, H, D_DEC = 128, 128, 128, 512
NODE_VOCAB = 512
N = B * C
NCTX = 2 * N
CHUNK_CTX = 16
ROWS_PER_CHUNK = CHUNK_CTX * S
N_CHUNKS = NCTX // CHUNK_CTX
NW = 32
CHUNKS_PER_W = N_CHUNKS // NW
LANES = 16
MROW = ROWS_PER_CHUNK * LANES
TILE = 512
G4 = 4 * H


def _sc_pool_body(idx_hbm, mask_hbm, table_hbm, out_hbm,
                  idx_all, mask_v, rows_v, acc_v, gsem, msem, osem):
    wid = lax.axis_index("s") * 2 + lax.axis_index("c")
    base = wid * CHUNKS_PER_W

    pltpu.sync_copy(idx_hbm.at[wid], idx_all)

    def issue(k, b):
        pltpu.async_copy(mask_hbm.at[base + k], mask_v.at[b], msem.at[b])
        pltpu.async_copy(
            table_hbm.at[idx_all.at[pl.ds(k * ROWS_PER_CHUNK, ROWS_PER_CHUNK)]],
            rows_v.at[b], gsem.at[b])

    issue(0, 0)
    issue(1, 1)

    def outer(jj, _):
        for b in range(2):
            k = 2 * jj + b
            pltpu.make_async_copy(mask_hbm.at[0], mask_v.at[b], msem.at[b]).wait()
            pltpu.make_async_copy(
                table_hbm.at[idx_all.at[pl.ds(0, ROWS_PER_CHUNK)]],
                rows_v.at[b], gsem.at[b]).wait()

            @pl.when(k >= 2)
            def _():
                pltpu.make_async_copy(
                    acc_v.at[b], out_hbm.at[pl.ds(0, CHUNK_CTX)], osem.at[b]).wait()

            for ci in range(CHUNK_CTX):
                ms = [mask_v[b, pl.ds((ci * S + s) * LANES, LANES)] for s in range(S)]
                for v in range(D_TOK // LANES):
                    acc = rows_v[b, ci * S + 0, pl.ds(v * LANES, LANES)] * ms[0]
                    for s in range(1, S):
                        acc = acc + rows_v[b, ci * S + s, pl.ds(v * LANES, LANES)] * ms[s]
                    acc_v[b, ci, pl.ds(v * LANES, LANES)] = acc
            pltpu.async_copy(
                acc_v.at[b], out_hbm.at[pl.ds((base + k) * CHUNK_CTX, CHUNK_CTX)],
                osem.at[b])

            @pl.when(k + 2 < CHUNKS_PER_W)
            def _():
                issue(k + 2, b)
        return 0

    lax.fori_loop(0, CHUNKS_PER_W // 2, outer, 0)
    for b in range(2):
        pltpu.make_async_copy(
            acc_v.at[b], out_hbm.at[pl.ds(0, CHUNK_CTX)], osem.at[b]).wait()


def _sc_pool_call(idx_cat, mask_cat, table):
    mesh = plsc.VectorSubcoreMesh(core_axis_name="c", subcore_axis_name="s")
    fn = functools.partial(
        pl.kernel,
        mesh=mesh,
        out_type=jax.ShapeDtypeStruct((NCTX, D_TOK), jnp.float32),
        scratch_types=[
            pltpu.VMEM((CHUNKS_PER_W * ROWS_PER_CHUNK,), jnp.int32),
            pltpu.VMEM((2, MROW), jnp.float32),
            pltpu.VMEM((2, ROWS_PER_CHUNK, D_TOK), jnp.float32),
            pltpu.VMEM((2, CHUNK_CTX, D_TOK), jnp.float32),
            pltpu.SemaphoreType.DMA((2,)),
            pltpu.SemaphoreType.DMA((2,)),
            pltpu.SemaphoreType.DMA((2,)),
        ],
    )(_sc_pool_body)
    return fn(idx_cat, mask_cat, table)


def _prep_body(emb_ref, wxf_ref, bf_ref, wxb_ref, bb_ref, whf_ref, whb_ref,
               emb_o, wf_ref, wb_ref, bias_ref):
    f32 = jnp.float32
    bf16 = jnp.bfloat16
    lane = lax.broadcasted_iota(jnp.int32, (1, G4), 1)
    sc = jnp.where((lane >= 2 * H) & (lane < 3 * H), 1.0, 0.5).astype(f32)
    emb_o[...] = emb_ref[...].astype(bf16)
    wf_ref[0:H, :] = (wxf_ref[...] * sc).astype(bf16)
    wf_ref[H:2 * H, :] = (whf_ref[...] * sc).astype(bf16)
    wb_ref[0:H, :] = (wxb_ref[...] * sc).astype(bf16)
    wb_ref[H:2 * H, :] = (whb_ref[...] * sc).astype(bf16)
    bias_ref[0:1, :] = bf_ref[...] * sc
    bias_ref[1:2, :] = bb_ref[...] * sc


def _prep_call(node_embedding, Wx_f, b_f, Wx_b, b_b, Wh_f, Wh_b):
    return pl.pallas_call(
        _prep_body,
        out_shape=(
            jax.ShapeDtypeStruct((NODE_VOCAB, H), jnp.bfloat16),
            jax.ShapeDtypeStruct((2 * H, G4), jnp.bfloat16),
            jax.ShapeDtypeStruct((2 * H, G4), jnp.bfloat16),
            jax.ShapeDtypeStruct((2, G4), jnp.float32),
        ),
    )(node_embedding, Wx_f, b_f.reshape(1, G4), Wx_b, b_b.reshape(1, G4),
      Wh_f, Wh_b)


def _lstm_body(idx_ref, len_ref, emb_ref, wf_ref, wb_ref, bias_ref, out_ref, x_s):
    f32 = jnp.float32
    bf16 = jnp.bfloat16
    idx = idx_ref[...]
    iota = lax.broadcasted_iota(jnp.int32, (TILE, NODE_VOCAB), 1)
    lenc = len_ref[...]
    emb = emb_ref[...]
    wf = wf_ref[...]
    wb = wb_ref[...]
    bias_f = bias_ref[0:1, :]
    bias_b = bias_ref[1:2, :]
    hf = jnp.zeros((TILE, H), f32)
    cf = jnp.zeros((TILE, H), f32)
    hb = jnp.zeros((TILE, H), f32)
    cb = jnp.zeros((TILE, H), f32)

    for p in range(L):
        oh = (idx[:, p:p + 1] == iota).astype(bf16)
        x_s[pl.ds(p * TILE, TILE), :] = jnp.dot(
            oh, emb, preferred_element_type=f32).astype(bf16)

    def cell(gate, h, c, mask):
        ti = jnp.tanh(gate[:, 0:H])
        tf_ = jnp.tanh(gate[:, H:2 * H])
        g = jnp.tanh(gate[:, 2 * H:3 * H])
        to = jnp.tanh(gate[:, 3 * H:4 * H])
        c_new = 0.5 * ((c + g) + (tf_ * c + ti * g))
        tc = jnp.tanh(c_new)
        h_new = 0.5 * (tc + to * tc)
        return jnp.where(mask, h_new, h), jnp.where(mask, c_new, c)

    for t in range(L):
        xf = x_s[pl.ds(t * TILE, TILE), :]
        xb = x_s[pl.ds((L - 1 - t) * TILE, TILE), :]
        mf = jnp.concatenate([xf, hf.astype(bf16)], axis=1)
        mb = jnp.concatenate([xb, hb.astype(bf16)], axis=1)
        gf = jnp.dot(mf, wf, preferred_element_type=f32) + bias_f
        gb = jnp.dot(mb, wb, preferred_element_type=f32) + bias_b
        hf, cf = cell(gf, hf, cf, t < lenc)
        hb, cb = cell(gb, hb, cb, (L - 1 - t) < lenc)

    out_ref[...] = jnp.concatenate([hf, hb], axis=1).astype(bf16)


def _lstm_call(idx, lens, emb, wf, wb, bias):
    row = lambda i: (i, 0)
    rep = lambda i: (0, 0)
    return pl.pallas_call(
        _lstm_body,
        grid=(N // TILE,),
        in_specs=[
            pl.BlockSpec((TILE, L), row),
            pl.BlockSpec((TILE, 1), row),
            pl.BlockSpec((NODE_VOCAB, H), rep),
            pl.BlockSpec((2 * H, G4), rep),
            pl.BlockSpec((2 * H, G4), rep),
            pl.BlockSpec((2, G4), rep),
        ],
        out_specs=pl.BlockSpec((TILE, 2 * H), row),
        out_shape=jax.ShapeDtypeStruct((N, 2 * H), jnp.bfloat16),
        scratch_shapes=[
            pltpu.VMEM((L * TILE, H), jnp.bfloat16),
        ],
    )(idx, lens, emb, wf, wb, bias)


def _gemm_body(sa_ref, ta_ref, h_ref, cvm_ref, w_ref, out_ref):
    f32 = jnp.float32
    bf16 = jnp.bfloat16
    w = w_ref[...]
    cvm = cvm_ref[...].astype(bf16)
    out = (jnp.dot(sa_ref[...].astype(bf16), w[0:D_TOK], preferred_element_type=f32)
           + jnp.dot(h_ref[...] * cvm, w[D_TOK:D_TOK + 2 * H], preferred_element_type=f32)
           + jnp.dot(ta_ref[...].astype(bf16), w[D_TOK + 2 * H:], preferred_element_type=f32))
    out_ref[...] = jnp.tanh(out)


def _gemm_call(src_agg, tgt_agg, hcat, cvm, wctx):
    row = lambda i: (i, 0)
    rep = lambda i: (0, 0)
    return pl.pallas_call(
        _gemm_body,
        grid=(N // TILE,),
        in_specs=[
            pl.BlockSpec((TILE, D_TOK), row),
            pl.BlockSpec((TILE, D_TOK), row),
            pl.BlockSpec((TILE, 2 * H), row),
            pl.BlockSpec((TILE, 1), row),
            pl.BlockSpec((2 * (D_TOK + H), D_DEC), rep),
        ],
        out_specs=pl.BlockSpec((TILE, D_DEC), row),
        out_shape=jax.ShapeDtypeStruct((N, D_DEC), jnp.float32),
    )(src_agg, tgt_agg, hcat, cvm, wctx)


def kernel(source_subtoken_indices, node_indices, target_subtoken_indices,
           source_subtoken_lengths, node_lengths, target_subtoken_lengths,
           context_valid_mask, subtoken_embedding, node_embedding,
           Wx_f, Wh_f, b_f, Wx_b, Wh_b, b_b, W_ctx):
    src_idx = source_subtoken_indices.reshape(N, S)
    tgt_idx = target_subtoken_indices.reshape(N, S)
    idx_cat = jnp.concatenate([src_idx, tgt_idx], axis=0).reshape(
        NW, CHUNKS_PER_W * ROWS_PER_CHUNK)
    ar = jnp.arange(S)[None, :]
    src_mask = (ar < source_subtoken_lengths.reshape(N, 1)).astype(jnp.float32)
    tgt_mask = (ar < target_subtoken_lengths.reshape(N, 1)).astype(jnp.float32)
    mask_cat = jnp.concatenate([src_mask, tgt_mask], axis=0).reshape(NCTX * S, 1)
    mask_cat = jnp.broadcast_to(mask_cat, (NCTX * S, LANES)).reshape(N_CHUNKS, MROW)

    nidx = node_indices.reshape(N, L)
    lens = node_lengths.reshape(N, 1)
    wctx = W_ctx.astype(jnp.bfloat16)

    pooled = _sc_pool_call(idx_cat, mask_cat, subtoken_embedding)

    embb, wf, wb, bias = _prep_call(node_embedding, Wx_f, b_f, Wx_b, b_b, Wh_f, Wh_b)
    hcat = _lstm_call(nidx, lens, embb, wf, wb, bias)
    out = _gemm_call(pooled[:N], pooled[N:], hcat,
                     context_valid_mask.reshape(N, 1), wctx)
    return out.reshape(B, C, D_DEC)


# speedup vs baseline: 1.9190x; 1.1540x over previous
import functools

import jax
import jax.numpy as jnp
from jax import lax
from jax.experimental import pallas as pl
from jax.experimental.pallas import tpu as pltpu
from jax.experimental.pallas import tpu_sc as plsc

B, C, S, L = 64, 200, 6, 9
D_TOK, D_NODE, H, D_DEC = 128, 128, 128, 512
NODE_VOCAB = 512
N = B * C
NCTX = 2 * N
CHUNK_CTX = 16
ROWS_PER_CHUNK = CHUNK_CTX * S
N_CHUNKS = NCTX // CHUNK_CTX
NW = 32
CHUNKS_PER_W = N_CHUNKS // NW
LANES = 16
MROW = ROWS_PER_CHUNK * LANES
TILE = 512
G4 = 4 * H


def _sc_pool_body(idx_hbm, mask_hbm, table_hbm, out_hbm,
                  idx_all, mask_v, rows_v, acc_v, gsem, msem, osem):
    wid = lax.axis_index("s") * 2 + lax.axis_index("c")
    base = wid * CHUNKS_PER_W

    pltpu.sync_copy(idx_hbm.at[wid], idx_all)

    def issue(k, b):
        pltpu.async_copy(mask_hbm.at[base + k], mask_v.at[b], msem.at[b])
        pltpu.async_copy(
            table_hbm.at[idx_all.at[pl.ds(k * ROWS_PER_CHUNK, ROWS_PER_CHUNK)]],
            rows_v.at[b], gsem.at[b])

    issue(0, 0)
    issue(1, 1)

    def outer(jj, _):
        for b in range(2):
            k = 2 * jj + b
            pltpu.make_async_copy(mask_hbm.at[0], mask_v.at[b], msem.at[b]).wait()
            pltpu.make_async_copy(
                table_hbm.at[idx_all.at[pl.ds(0, ROWS_PER_CHUNK)]],
                rows_v.at[b], gsem.at[b]).wait()

            @pl.when(k >= 2)
            def _():
                pltpu.make_async_copy(
                    acc_v.at[b], out_hbm.at[pl.ds(0, CHUNK_CTX)], osem.at[b]).wait()

            for ci in range(CHUNK_CTX):
                ms = [mask_v[b, pl.ds((ci * S + s) * LANES, LANES)] for s in range(S)]
                for v in range(D_TOK // LANES):
                    acc = rows_v[b, ci * S + 0, pl.ds(v * LANES, LANES)] * ms[0]
                    for s in range(1, S):
                        acc = acc + rows_v[b, ci * S + s, pl.ds(v * LANES, LANES)] * ms[s]
                    acc_v[b, ci, pl.ds(v * LANES, LANES)] = acc
            pltpu.async_copy(
                acc_v.at[b], out_hbm.at[pl.ds((base + k) * CHUNK_CTX, CHUNK_CTX)],
                osem.at[b])

            @pl.when(k + 2 < CHUNKS_PER_W)
            def _():
                issue(k + 2, b)
        return 0

    lax.fori_loop(0, CHUNKS_PER_W // 2, outer, 0)
    for b in range(2):
        pltpu.make_async_copy(
            acc_v.at[b], out_hbm.at[pl.ds(0, CHUNK_CTX)], osem.at[b]).wait()


def _sc_pool_call(idx_cat, mask_cat, table):
    mesh = plsc.VectorSubcoreMesh(core_axis_name="c", subcore_axis_name="s")
    fn = functools.partial(
        pl.kernel,
        mesh=mesh,
        out_type=jax.ShapeDtypeStruct((NCTX, D_TOK), jnp.float32),
        scratch_types=[
            pltpu.VMEM((CHUNKS_PER_W * ROWS_PER_CHUNK,), jnp.int32),
            pltpu.VMEM((2, MROW), jnp.float32),
            pltpu.VMEM((2, ROWS_PER_CHUNK, D_TOK), jnp.float32),
            pltpu.VMEM((2, CHUNK_CTX, D_TOK), jnp.float32),
            pltpu.SemaphoreType.DMA((2,)),
            pltpu.SemaphoreType.DMA((2,)),
            pltpu.SemaphoreType.DMA((2,)),
        ],
    )(_sc_pool_body)
    return fn(idx_cat, mask_cat, table)


def _prep_body(emb_ref, wxf_ref, bf_ref, wxb_ref, bb_ref, whf_ref, whb_ref,
               lens_ref, emb_o, wf_ref, wb_ref, bias_ref, mask_ref):
    f32 = jnp.float32
    bf16 = jnp.bfloat16
    lane = lax.broadcasted_iota(jnp.int32, (1, G4), 1)
    sc = jnp.where((lane >= 2 * H) & (lane < 3 * H), 1.0, 0.5).astype(f32)
    emb_o[...] = emb_ref[...].astype(bf16)
    wf_ref[0:H, :] = (wxf_ref[...] * sc).astype(bf16)
    wf_ref[H:2 * H, :] = (whf_ref[...] * sc).astype(bf16)
    wb_ref[0:H, :] = (wxb_ref[...] * sc).astype(bf16)
    wb_ref[H:2 * H, :] = (whb_ref[...] * sc).astype(bf16)
    bias_ref[0:1, :] = bf_ref[...] * sc
    bias_ref[1:2, :] = bb_ref[...] * sc

    ci_of = lax.broadcasted_iota(jnp.int32, (CHUNK_CTX, MROW), 1) // (S * LANES)
    ci_row = lax.broadcasted_iota(jnp.int32, (CHUNK_CTX, MROW), 0)
    sel = (ci_of == ci_row).astype(f32)
    lens_e = jnp.dot(lens_ref[...], sel, preferred_element_type=f32)
    s_of = ((lax.broadcasted_iota(jnp.int32, (1, MROW), 1) // LANES) % S).astype(f32)
    mask_ref[...] = (s_of < lens_e).astype(f32)


def _prep_call(node_embedding, Wx_f, b_f, Wx_b, b_b, Wh_f, Wh_b, lens_pool):
    return pl.pallas_call(
        _prep_body,
        out_shape=(
            jax.ShapeDtypeStruct((NODE_VOCAB, H), jnp.bfloat16),
            jax.ShapeDtypeStruct((2 * H, G4), jnp.bfloat16),
            jax.ShapeDtypeStruct((2 * H, G4), jnp.bfloat16),
            jax.ShapeDtypeStruct((2, G4), jnp.float32),
            jax.ShapeDtypeStruct((N_CHUNKS, MROW), jnp.float32),
        ),
    )(node_embedding, Wx_f, b_f.reshape(1, G4), Wx_b, b_b.reshape(1, G4),
      Wh_f, Wh_b, lens_pool)


def _lstm_body(idx_ref, len_ref, emb_ref, wf_ref, wb_ref, bias_ref, out_ref, x_s):
    f32 = jnp.float32
    bf16 = jnp.bfloat16
    idx = idx_ref[...]
    iota = lax.broadcasted_iota(jnp.int32, (TILE, NODE_VOCAB), 1)
    lenc = len_ref[...]
    emb = emb_ref[...]
    wf = wf_ref[...]
    wb = wb_ref[...]
    bias_f = bias_ref[0:1, :]
    bias_b = bias_ref[1:2, :]
    hf = jnp.zeros((TILE, H), f32)
    cf = jnp.zeros((TILE, H), f32)
    hb = jnp.zeros((TILE, H), f32)
    cb = jnp.zeros((TILE, H), f32)

    for p in range(L):
        oh = (idx[:, p:p + 1] == iota).astype(bf16)
        x_s[pl.ds(p * TILE, TILE), :] = jnp.dot(
            oh, emb, preferred_element_type=f32).astype(bf16)

    def cell(gate, h, c, mask):
        ti = jnp.tanh(gate[:, 0:H])
        tf_ = jnp.tanh(gate[:, H:2 * H])
        g = jnp.tanh(gate[:, 2 * H:3 * H])
        to = jnp.tanh(gate[:, 3 * H:4 * H])
        c_new = 0.5 * ((c + g) + (tf_ * c + ti * g))
        tc = jnp.tanh(c_new)
        h_new = 0.5 * (tc + to * tc)
        return jnp.where(mask, h_new, h), jnp.where(mask, c_new, c)

    for t in range(L):
        xf = x_s[pl.ds(t * TILE, TILE), :]
        xb = x_s[pl.ds((L - 1 - t) * TILE, TILE), :]
        mf = jnp.concatenate([xf, hf.astype(bf16)], axis=1)
        mb = jnp.concatenate([xb, hb.astype(bf16)], axis=1)
        gf = jnp.dot(mf, wf, preferred_element_type=f32) + bias_f
        gb = jnp.dot(mb, wb, preferred_element_type=f32) + bias_b
        hf, cf = cell(gf, hf, cf, t < lenc)
        hb, cb = cell(gb, hb, cb, (L - 1 - t) < lenc)

    out_ref[...] = jnp.concatenate([hf, hb], axis=1).astype(bf16)


def _lstm_call(idx, lens, emb, wf, wb, bias):
    row = lambda i: (i, 0)
    rep = lambda i: (0, 0)
    return pl.pallas_call(
        _lstm_body,
        grid=(N // TILE,),
        in_specs=[
            pl.BlockSpec((TILE, L), row),
            pl.BlockSpec((TILE, 1), row),
            pl.BlockSpec((NODE_VOCAB, H), rep),
            pl.BlockSpec((2 * H, G4), rep),
            pl.BlockSpec((2 * H, G4), rep),
            pl.BlockSpec((2, G4), rep),
        ],
        out_specs=pl.BlockSpec((TILE, 2 * H), row),
        out_shape=jax.ShapeDtypeStruct((N, 2 * H), jnp.bfloat16),
        scratch_shapes=[
            pltpu.VMEM((L * TILE, H), jnp.bfloat16),
        ],
    )(idx, lens, emb, wf, wb, bias)


def _gemm_body(sa_ref, ta_ref, h_ref, cvm_ref, w_ref, out_ref):
    f32 = jnp.float32
    bf16 = jnp.bfloat16
    w = w_ref[...]
    cvm = cvm_ref[...].astype(bf16)
    out = (jnp.dot(sa_ref[...].astype(bf16), w[0:D_TOK], preferred_element_type=f32)
           + jnp.dot(h_ref[...] * cvm, w[D_TOK:D_TOK + 2 * H], preferred_element_type=f32)
           + jnp.dot(ta_ref[...].astype(bf16), w[D_TOK + 2 * H:], preferred_element_type=f32))
    out_ref[...] = jnp.tanh(out)


def _gemm_call(pooled, hcat, cvm, wctx):
    row = lambda i: (i, 0)
    rep = lambda i: (0, 0)
    tgt_row = lambda i: (i + N // TILE, 0)
    return pl.pallas_call(
        _gemm_body,
        grid=(N // TILE,),
        in_specs=[
            pl.BlockSpec((TILE, D_TOK), row),
            pl.BlockSpec((TILE, D_TOK), tgt_row),
            pl.BlockSpec((TILE, 2 * H), row),
            pl.BlockSpec((TILE, 1), row),
            pl.BlockSpec((2 * (D_TOK + H), D_DEC), rep),
        ],
        out_specs=pl.BlockSpec((TILE, D_DEC), row),
        out_shape=jax.ShapeDtypeStruct((N, D_DEC), jnp.float32),
    )(pooled, pooled, hcat, cvm, wctx)


def kernel(source_subtoken_indices, node_indices, target_subtoken_indices,
           source_subtoken_lengths, node_lengths, target_subtoken_lengths,
           context_valid_mask, subtoken_embedding, node_embedding,
           Wx_f, Wh_f, b_f, Wx_b, Wh_b, b_b, W_ctx):
    src_idx = source_subtoken_indices.reshape(N, S)
    tgt_idx = target_subtoken_indices.reshape(N, S)
    idx_cat = jnp.concatenate([src_idx, tgt_idx], axis=0).reshape(
        NW, CHUNKS_PER_W * ROWS_PER_CHUNK)
    lens_pool = jnp.concatenate(
        [source_subtoken_lengths.reshape(N), target_subtoken_lengths.reshape(N)]
    ).astype(jnp.float32).reshape(N_CHUNKS, CHUNK_CTX)

    nidx = node_indices.reshape(N, L)
    lens = node_lengths.reshape(N, 1)
    wctx = W_ctx.astype(jnp.bfloat16)

    embb, wf, wb, bias, mask_cat = _prep_call(
        node_embedding, Wx_f, b_f, Wx_b, b_b, Wh_f, Wh_b, lens_pool)

    pooled = _sc_pool_call(idx_cat, mask_cat, subtoken_embedding)

    hcat = _lstm_call(nidx, lens, embb, wf, wb, bias)
    out = _gemm_call(pooled, hcat, context_valid_mask.reshape(N, 1), wctx)
    return out.reshape(B, C, D_DEC)


# speedup vs baseline: 1.9383x; 1.0101x over previous
import functools

import jax
import jax.numpy as jnp
from jax import lax
from jax.experimental import pallas as pl
from jax.experimental.pallas import tpu as pltpu
from jax.experimental.pallas import tpu_sc as plsc

B, C, S, L = 64, 200, 6, 9
D_TOK, D_NODE, H, D_DEC = 128, 128, 128, 512
NODE_VOCAB = 512
N = B * C
NCTX = 2 * N
CHUNK_CTX = 16
ROWS_PER_CHUNK = CHUNK_CTX * S
N_CHUNKS = NCTX // CHUNK_CTX
NW = 32
CHUNKS_PER_W = N_CHUNKS // NW
LANES = 16
MROW = ROWS_PER_CHUNK * LANES
TILE = 512
G4 = 4 * H


def _sc_pool_body(idx_hbm, mask_hbm, table_hbm, out_hbm,
                  idx_all, mask_v, rows_v, acc_v, gsem, msem, osem):
    wid = lax.axis_index("s") * 2 + lax.axis_index("c")
    base = wid * CHUNKS_PER_W

    pltpu.sync_copy(idx_hbm.at[wid], idx_all)

    def issue(k, b):
        pltpu.async_copy(mask_hbm.at[base + k], mask_v.at[b], msem.at[b])
        pltpu.async_copy(
            table_hbm.at[idx_all.at[pl.ds(k * ROWS_PER_CHUNK, ROWS_PER_CHUNK)]],
            rows_v.at[b], gsem.at[b])

    issue(0, 0)
    issue(1, 1)

    def outer(jj, _):
        for b in range(2):
            k = 2 * jj + b
            pltpu.make_async_copy(mask_hbm.at[0], mask_v.at[b], msem.at[b]).wait()
            pltpu.make_async_copy(
                table_hbm.at[idx_all.at[pl.ds(0, ROWS_PER_CHUNK)]],
                rows_v.at[b], gsem.at[b]).wait()

            @pl.when(k >= 2)
            def _():
                pltpu.make_async_copy(
                    acc_v.at[b], out_hbm.at[pl.ds(0, CHUNK_CTX)], osem.at[b]).wait()

            for ci in range(CHUNK_CTX):
                ms = [mask_v[b, pl.ds((ci * S + s) * LANES, LANES)] for s in range(S)]
                for v in range(D_TOK // LANES):
                    acc = rows_v[b, ci * S + 0, pl.ds(v * LANES, LANES)] * ms[0]
                    for s in range(1, S):
                        acc = acc + rows_v[b, ci * S + s, pl.ds(v * LANES, LANES)] * ms[s]
                    acc_v[b, ci, pl.ds(v * LANES, LANES)] = acc
            pltpu.async_copy(
                acc_v.at[b], out_hbm.at[pl.ds((base + k) * CHUNK_CTX, CHUNK_CTX)],
                osem.at[b])

            @pl.when(k + 2 < CHUNKS_PER_W)
            def _():
                issue(k + 2, b)
        return 0

    lax.fori_loop(0, CHUNKS_PER_W // 2, outer, 0)
    for b in range(2):
        pltpu.make_async_copy(
            acc_v.at[b], out_hbm.at[pl.ds(0, CHUNK_CTX)], osem.at[b]).wait()


def _sc_pool_call(idx_cat, mask_cat, table):
    mesh = plsc.VectorSubcoreMesh(core_axis_name="c", subcore_axis_name="s")
    fn = functools.partial(
        pl.kernel,
        mesh=mesh,
        out_type=jax.ShapeDtypeStruct((NCTX, D_TOK), jnp.float32),
        scratch_types=[
            pltpu.VMEM((CHUNKS_PER_W * ROWS_PER_CHUNK,), jnp.int32),
            pltpu.VMEM((2, MROW), jnp.float32),
            pltpu.VMEM((2, ROWS_PER_CHUNK, D_TOK), jnp.float32),
            pltpu.VMEM((2, CHUNK_CTX, D_TOK), jnp.float32),
            pltpu.SemaphoreType.DMA((2,)),
            pltpu.SemaphoreType.DMA((2,)),
            pltpu.SemaphoreType.DMA((2,)),
        ],
    )(_sc_pool_body)
    return fn(idx_cat, mask_cat, table)


def _prep_body(emb_ref, wxf_ref, bf_ref, wxb_ref, bb_ref, whf_ref, whb_ref,
               lens_ref, emb_o, wf_ref, wb_ref, bias_ref, mask_ref):
    f32 = jnp.float32
    bf16 = jnp.bfloat16
    lane = lax.broadcasted_iota(jnp.int32, (1, G4), 1)
    sc = jnp.where((lane >= 2 * H) & (lane < 3 * H), 1.0, 0.5).astype(f32)
    emb_o[...] = emb_ref[...].astype(bf16)
    wf_ref[0:H, :] = (wxf_ref[...] * sc).astype(bf16)
    wf_ref[H:2 * H, :] = (whf_ref[...] * sc).astype(bf16)
    wb_ref[0:H, :] = (wxb_ref[...] * sc).astype(bf16)
    wb_ref[H:2 * H, :] = (whb_ref[...] * sc).astype(bf16)
    bias_ref[0:1, :] = bf_ref[...] * sc
    bias_ref[1:2, :] = bb_ref[...] * sc

    ci_of = lax.broadcasted_iota(jnp.int32, (CHUNK_CTX, MROW), 1) // (S * LANES)
    ci_row = lax.broadcasted_iota(jnp.int32, (CHUNK_CTX, MROW), 0)
    sel = (ci_of == ci_row).astype(f32)
    lens_e = jnp.dot(lens_ref[...], sel, preferred_element_type=f32)
    s_of = ((lax.broadcasted_iota(jnp.int32, (1, MROW), 1) // LANES) % S).astype(f32)
    mask_ref[...] = (s_of < lens_e).astype(f32)


def _prep_call(node_embedding, Wx_f, b_f, Wx_b, b_b, Wh_f, Wh_b, lens_pool):
    return pl.pallas_call(
        _prep_body,
        out_shape=(
            jax.ShapeDtypeStruct((NODE_VOCAB, H), jnp.bfloat16),
            jax.ShapeDtypeStruct((2 * H, G4), jnp.bfloat16),
            jax.ShapeDtypeStruct((2 * H, G4), jnp.bfloat16),
            jax.ShapeDtypeStruct((2, G4), jnp.float32),
            jax.ShapeDtypeStruct((N_CHUNKS, MROW), jnp.float32),
        ),
    )(node_embedding, Wx_f, b_f.reshape(1, G4), Wx_b, b_b.reshape(1, G4),
      Wh_f, Wh_b, lens_pool)


def _lstm_body(idxl_ref, emb_ref, wf_ref, wb_ref, bias_ref, out_ref, x_s):
    f32 = jnp.float32
    bf16 = jnp.bfloat16
    idxl = idxl_ref[...]
    idx = lax.transpose(idxl, (1, 0))
    iota = lax.broadcasted_iota(jnp.int32, (TILE, NODE_VOCAB), 1)
    lenc = idx[:, L:L + 1]
    emb = emb_ref[...]
    wf = wf_ref[...]
    wb = wb_ref[...]
    bias_f = bias_ref[0:1, :]
    bias_b = bias_ref[1:2, :]
    hf = jnp.zeros((TILE, H), f32)
    cf = jnp.zeros((TILE, H), f32)
    hb = jnp.zeros((TILE, H), f32)
    cb = jnp.zeros((TILE, H), f32)

    for p in range(L):
        oh = (idx[:, p:p + 1] == iota).astype(bf16)
        x_s[pl.ds(p * TILE, TILE), :] = jnp.dot(
            oh, emb, preferred_element_type=f32).astype(bf16)

    def cell(gate, h, c, mask):
        ti = jnp.tanh(gate[:, 0:H])
        tf_ = jnp.tanh(gate[:, H:2 * H])
        g = jnp.tanh(gate[:, 2 * H:3 * H])
        to = jnp.tanh(gate[:, 3 * H:4 * H])
        c_new = 0.5 * ((c + g) + (tf_ * c + ti * g))
        tc = jnp.tanh(c_new)
        h_new = 0.5 * (tc + to * tc)
        return jnp.where(mask, h_new, h), jnp.where(mask, c_new, c)

    for t in range(L):
        xf = x_s[pl.ds(t * TILE, TILE), :]
        xb = x_s[pl.ds((L - 1 - t) * TILE, TILE), :]
        mf = jnp.concatenate([xf, hf.astype(bf16)], axis=1)
        mb = jnp.concatenate([xb, hb.astype(bf16)], axis=1)
        gf = jnp.dot(mf, wf, preferred_element_type=f32) + bias_f
        gb = jnp.dot(mb, wb, preferred_element_type=f32) + bias_b
        hf, cf = cell(gf, hf, cf, t < lenc)
        hb, cb = cell(gb, hb, cb, (L - 1 - t) < lenc)

    out_ref[...] = jnp.concatenate([hf, hb], axis=1).astype(bf16)


def _lstm_call(idxl, emb, wf, wb, bias):
    row = lambda i: (i, 0)
    rep = lambda i: (0, 0)
    return pl.pallas_call(
        _lstm_body,
        grid=(N // TILE,),
        in_specs=[
            pl.BlockSpec((L + 1, TILE), lambda i: (0, i)),
            pl.BlockSpec((NODE_VOCAB, H), rep),
            pl.BlockSpec((2 * H, G4), rep),
            pl.BlockSpec((2 * H, G4), rep),
            pl.BlockSpec((2, G4), rep),
        ],
        out_specs=pl.BlockSpec((TILE, 2 * H), row),
        out_shape=jax.ShapeDtypeStruct((N, 2 * H), jnp.bfloat16),
        scratch_shapes=[
            pltpu.VMEM((L * TILE, H), jnp.bfloat16),
        ],
    )(idxl, emb, wf, wb, bias)


def _gemm_body(sa_ref, ta_ref, h_ref, cvm_ref, w_ref, out_ref):
    f32 = jnp.float32
    bf16 = jnp.bfloat16
    w = w_ref[...]
    cvm = cvm_ref[...].astype(bf16)
    out = (jnp.dot(sa_ref[...].astype(bf16), w[0:D_TOK], preferred_element_type=f32)
           + jnp.dot(h_ref[...] * cvm, w[D_TOK:D_TOK + 2 * H], preferred_element_type=f32)
           + jnp.dot(ta_ref[...].astype(bf16), w[D_TOK + 2 * H:], preferred_element_type=f32))
    out_ref[...] = jnp.tanh(out)


def _gemm_call(pooled, hcat, cvm, wctx):
    row = lambda i: (i, 0)
    rep = lambda i: (0, 0)
    tgt_row = lambda i: (i + N // TILE, 0)
    return pl.pallas_call(
        _gemm_body,
        grid=(N // TILE,),
        in_specs=[
            pl.BlockSpec((TILE, D_TOK), row),
            pl.BlockSpec((TILE, D_TOK), tgt_row),
            pl.BlockSpec((TILE, 2 * H), row),
            pl.BlockSpec((TILE, 1), row),
            pl.BlockSpec((2 * (D_TOK + H), D_DEC), rep),
        ],
        out_specs=pl.BlockSpec((TILE, D_DEC), row),
        out_shape=jax.ShapeDtypeStruct((N, D_DEC), jnp.float32),
    )(pooled, pooled, hcat, cvm, wctx)


def kernel(source_subtoken_indices, node_indices, target_subtoken_indices,
           source_subtoken_lengths, node_lengths, target_subtoken_lengths,
           context_valid_mask, subtoken_embedding, node_embedding,
           Wx_f, Wh_f, b_f, Wx_b, Wh_b, b_b, W_ctx):
    src_idx = source_subtoken_indices.reshape(N, S)
    tgt_idx = target_subtoken_indices.reshape(N, S)
    idx_cat = jnp.concatenate([src_idx, tgt_idx], axis=0).reshape(
        NW, CHUNKS_PER_W * ROWS_PER_CHUNK)
    lens_pool = jnp.concatenate(
        [source_subtoken_lengths.reshape(N), target_subtoken_lengths.reshape(N)]
    ).astype(jnp.float32).reshape(N_CHUNKS, CHUNK_CTX)

    idxl = jnp.concatenate(
        [node_indices.reshape(N, L).T, node_lengths.reshape(1, N)], axis=0)
    wctx = W_ctx.astype(jnp.bfloat16)

    embb, wf, wb, bias, mask_cat = _prep_call(
        node_embedding, Wx_f, b_f, Wx_b, b_b, Wh_f, Wh_b, lens_pool)

    pooled = _sc_pool_call(idx_cat, mask_cat, subtoken_embedding)

    hcat = _lstm_call(idxl, embb, wf, wb, bias)
    out = _gemm_call(pooled, hcat, context_valid_mask.reshape(N, 1), wctx)
    return out.reshape(B, C, D_DEC)
